# Initial kernel scaffold; baseline (speedup 1.0000x reference)
import jax
import jax.numpy as jnp
from jax.experimental import pallas as pl


def kernel(h, edge_index, edge_relation, edge_norm, W, b):
    raise NotImplementedError("write your pallas kernel here")


# speedup vs baseline: 12.6828x; 12.6828x over previous
import functools

import jax
import jax.numpy as jnp
from jax import lax
from jax.experimental import pallas as pl
from jax.experimental.pallas import tpu as pltpu
from jax.experimental.pallas import tpu_sc as plsc

N = 10000
E = 320000
D_IN = 128
D_OUT = 128
NUM_REL = 4

NC = 2
NS = 16
LANES = 16

E_PER_TILE = E // (NC * NS)
META = 2000
CHUNK = 80
SUB = META // CHUNK
MCH = E_PER_TILE // META
NPAD = 10240
ROWS_T = NPAD // NS

_GATHER_DNUMS = lax.GatherDimensionNumbers(
    offset_dims=(), collapsed_slice_dims=(0,), start_index_map=(0,))


def _take16(v, idx16):
    return lax.gather(v, idx16[:, None], _GATHER_DNUMS, (1,),
                      mode=lax.GatherScatterMode.PROMISE_IN_BOUNDS)


def _mm_body(h_ref, wt_ref, o_ref):
    o_ref[0] = jnp.dot(h_ref[...], wt_ref[0],
                       preferred_element_type=jnp.float32)


def _hw_table(h, W):
    Wt = W.reshape(D_OUT, NUM_REL, D_IN).transpose(1, 2, 0)
    BN = 2000
    NB = N // BN
    return pl.pallas_call(
        _mm_body,
        grid=(NB, NUM_REL),
        in_specs=[
            pl.BlockSpec((BN, D_IN), lambda i, j: (i, 0)),
            pl.BlockSpec((1, D_IN, D_OUT), lambda i, j: (j, 0, 0)),
        ],
        out_specs=pl.BlockSpec((1, BN, D_OUT), lambda i, j: (j, i, 0)),
        out_shape=jax.ShapeDtypeStruct((NUM_REL, N, D_OUT), jnp.float32),
    )(h, Wt)


def _sc_body(hw, rowi, coli, reli, normi, out,
             acc, colb, relb, rowbL, normb, idxb, rowb, rowsb, zb, sem):
    c = lax.axis_index("c")
    s = lax.axis_index("s")

    zero16 = jnp.zeros((LANES,), jnp.float32)

    def zero_row(r, _):
        for j in range(D_OUT // LANES):
            zb[r, pl.ds(j * LANES, LANES)] = zero16
        return 0

    lax.fori_loop(0, CHUNK, zero_row, 0)
    for z in range(ROWS_T // CHUNK):
        pltpu.sync_copy(zb, acc.at[pl.ds(s * ROWS_T + z * CHUNK, CHUNK)])

    plsc.subcore_barrier()

    def outer(u, _):
        base = c * (NS * E_PER_TILE) + s * E_PER_TILE + u * META
        pltpu.sync_copy(coli.at[pl.ds(base, META)], colb)
        pltpu.sync_copy(reli.at[pl.ds(base, META)], relb)
        pltpu.sync_copy(rowi.at[pl.ds(base, META)], rowbL)
        pltpu.sync_copy(normi.at[pl.ds(base, META)], normb)

        def inner(t, _):
            off = t * CHUNK

            def mk_idx(g, _):
                sl_s = pl.ds(off + g * LANES, LANES)
                sl_d = pl.ds(g * LANES, LANES)
                idxb[sl_d] = colb[sl_s] + relb[sl_s] * N
                rowb[sl_d] = rowbL[sl_s]
                return 0

            lax.fori_loop(0, CHUNK // LANES, mk_idx, 0)

            pltpu.async_copy(hw.at[idxb], rowsb, sem).wait()

            def scale(g, _):
                nv = normb[pl.ds(off + g * LANES, LANES)]
                for i in range(LANES):
                    nspl = _take16(nv, jnp.full((LANES,), i, jnp.int32))
                    e = g * LANES + i
                    for j in range(D_OUT // LANES):
                        sl = pl.ds(j * LANES, LANES)
                        rowsb[e, sl] = rowsb[e, sl] * nspl
                return 0

            lax.fori_loop(0, CHUNK // LANES, scale, 0)

            pltpu.sync_copy(rowsb, acc.at[rowb], add=True)
            return 0

        lax.fori_loop(0, SUB, inner, 0)
        return 0

    lax.fori_loop(0, MCH, outer, 0)

    plsc.subcore_barrier()

    pltpu.sync_copy(acc.at[pl.ds(s * ROWS_T, ROWS_T)],
                    out.at[pl.ds(c * NPAD + s * ROWS_T, ROWS_T)])


_sc_agg = functools.partial(
    pl.kernel,
    out_type=jax.ShapeDtypeStruct((NC * NPAD, D_OUT), jnp.float32),
    mesh=plsc.VectorSubcoreMesh(core_axis_name="c", subcore_axis_name="s",
                                num_cores=NC, num_subcores=NS),
    scratch_types=[
        pltpu.VMEM_SHARED((NPAD, D_OUT), jnp.float32),
        pltpu.VMEM((META,), jnp.int32),
        pltpu.VMEM((META,), jnp.int32),
        pltpu.VMEM((META,), jnp.int32),
        pltpu.VMEM((META,), jnp.float32),
        pltpu.VMEM((CHUNK,), jnp.int32),
        pltpu.VMEM((CHUNK,), jnp.int32),
        pltpu.VMEM((CHUNK, D_OUT), jnp.float32),
        pltpu.VMEM((CHUNK, D_OUT), jnp.float32),
        pltpu.SemaphoreType.DMA,
    ],
)(_sc_body)


def _combine_body(p_ref, b_ref, o_ref):
    o_ref[...] = p_ref[0] + p_ref[1] + b_ref[...]


def _combine(parts, b):
    BN = 2048
    NB = NPAD // BN
    return pl.pallas_call(
        _combine_body,
        grid=(NB,),
        in_specs=[
            pl.BlockSpec((NC, BN, D_OUT), lambda i: (0, i, 0)),
            pl.BlockSpec((1, D_OUT), lambda i: (0, 0)),
        ],
        out_specs=pl.BlockSpec((BN, D_OUT), lambda i: (i, 0)),
        out_shape=jax.ShapeDtypeStruct((NPAD, D_OUT), jnp.float32),
    )(parts, b)


def kernel(h, edge_index, edge_relation, edge_norm, W, b):
    hw = _hw_table(h, W).reshape(NUM_REL * N, D_OUT)
    row = edge_index[0]
    col = edge_index[1]
    parts = _sc_agg(hw, row, col, edge_relation, edge_norm)
    out = _combine(parts.reshape(NC, NPAD, D_OUT), b.reshape(1, D_OUT))
    return out[:N]


# speedup vs baseline: 19.2482x; 1.5177x over previous
import functools

import jax
import jax.numpy as jnp
from jax import lax
from jax.experimental import pallas as pl
from jax.experimental.pallas import tpu as pltpu
from jax.experimental.pallas import tpu_sc as plsc

N = 10000
E = 320000
D_IN = 128
D_OUT = 128
NUM_REL = 4

NC = 2
NS = 16
LANES = 16

E_PER_TILE = E // (NC * NS)
META = 2000
CHUNK = 80
SUB = META // CHUNK
MCH = E_PER_TILE // META
NPAD = 10240
ROWS_T = NPAD // NS

_GATHER_DNUMS = lax.GatherDimensionNumbers(
    offset_dims=(), collapsed_slice_dims=(0,), start_index_map=(0,))


def _take16(v, idx16):
    return lax.gather(v, idx16[:, None], _GATHER_DNUMS, (1,),
                      mode=lax.GatherScatterMode.PROMISE_IN_BOUNDS)


def _mm_body(h_ref, wt_ref, o_ref):
    o_ref[0] = jnp.dot(h_ref[...], wt_ref[0],
                       preferred_element_type=jnp.float32)


def _hw_table(h, W):
    Wt = W.reshape(D_OUT, NUM_REL, D_IN).transpose(1, 2, 0)
    BN = 2000
    NB = N // BN
    return pl.pallas_call(
        _mm_body,
        grid=(NB, NUM_REL),
        in_specs=[
            pl.BlockSpec((BN, D_IN), lambda i, j: (i, 0)),
            pl.BlockSpec((1, D_IN, D_OUT), lambda i, j: (j, 0, 0)),
        ],
        out_specs=pl.BlockSpec((1, BN, D_OUT), lambda i, j: (j, i, 0)),
        out_shape=jax.ShapeDtypeStruct((NUM_REL, N, D_OUT), jnp.float32),
    )(h, Wt)


def _sc_body(hw, rowi, coli, reli, normi, out, acc,
             colb, relb, rowbL, normb,
             idxb0, idxb1, rowb0, rowb1, rowsb0, rowsb1, zb,
             gsem0, gsem1, ssem0, ssem1):
    c = lax.axis_index("c")
    s = lax.axis_index("s")
    idxb = (idxb0, idxb1)
    rowb = (rowb0, rowb1)
    rowsb = (rowsb0, rowsb1)
    gsem = (gsem0, gsem1)
    ssem = (ssem0, ssem1)

    zero16f = jnp.zeros((LANES,), jnp.float32)
    zero16i = jnp.zeros((LANES,), jnp.int32)

    def zero_row(r, _):
        for j in range(D_OUT // LANES):
            zb[r, pl.ds(j * LANES, LANES)] = zero16f
        return 0

    lax.fori_loop(0, CHUNK, zero_row, 0)
    for b in range(2):
        for g in range(CHUNK // LANES):
            rowb[b][pl.ds(g * LANES, LANES)] = zero16i + s * ROWS_T

    for z in range(ROWS_T // CHUNK):
        pltpu.sync_copy(zb, acc.at[pl.ds(s * ROWS_T + z * CHUNK, CHUNK)])

    for b in range(2):
        pltpu.async_copy(zb, acc.at[rowb[b]], ssem[b], add=True)

    plsc.subcore_barrier()

    tile_base = c * (NS * E_PER_TILE) + s * E_PER_TILE

    def mk_idx(q, b):
        def body(g, _):
            sl_s = pl.ds(q * CHUNK + g * LANES, LANES)
            sl_d = pl.ds(g * LANES, LANES)
            idxb[b][sl_d] = colb[sl_s] + relb[sl_s] * N
            rowb[b][sl_d] = rowbL[sl_s]
            return 0

        lax.fori_loop(0, CHUNK // LANES, body, 0)

    def scale(q, b):
        def body(g, _):
            nv = normb[pl.ds(q * CHUNK + g * LANES, LANES)]
            for i in range(LANES):
                nspl = _take16(nv, jnp.full((LANES,), i, jnp.int32))
                e = g * LANES + i
                for j in range(D_OUT // LANES):
                    sl = pl.ds(j * LANES, LANES)
                    rowsb[b][e, sl] = rowsb[b][e, sl] * nspl
            return 0

        lax.fori_loop(0, CHUNK // LANES, body, 0)

    def wait_scatter(b):
        pltpu.make_async_copy(rowsb[b], acc.at[rowb[b]], ssem[b]).wait()

    def issue_gather(b):
        pltpu.async_copy(hw.at[idxb[b]], rowsb[b], gsem[b])

    def wait_gather(b):
        pltpu.make_async_copy(hw.at[idxb[b]], rowsb[b], gsem[b]).wait()

    def issue_scatter(b):
        pltpu.async_copy(rowsb[b], acc.at[rowb[b]], ssem[b], add=True)

    def block(u, _):
        mb = tile_base + u * META
        pltpu.sync_copy(coli.at[pl.ds(mb, META)], colb)
        pltpu.sync_copy(reli.at[pl.ds(mb, META)], relb)
        pltpu.sync_copy(rowi.at[pl.ds(mb, META)], rowbL)
        pltpu.sync_copy(normi.at[pl.ds(mb, META)], normb)

        wait_scatter(0)
        mk_idx(0, 0)
        issue_gather(0)

        def pair(k, _):
            for (qo, cur) in ((0, 0), (1, 1)):
                q = 2 * k + qo
                oth = 1 - cur
                wait_scatter(oth)
                mk_idx(q + 1, oth)
                issue_gather(oth)
                wait_gather(cur)
                scale(q, cur)
                issue_scatter(cur)
            return 0

        lax.fori_loop(0, (SUB - 1) // 2, pair, 0)

        wait_gather(0)
        scale(SUB - 1, 0)
        issue_scatter(0)
        return 0

    lax.fori_loop(0, MCH, block, 0)

    wait_scatter(1)
    wait_scatter(0)

    plsc.subcore_barrier()

    pltpu.sync_copy(acc.at[pl.ds(s * ROWS_T, ROWS_T)],
                    out.at[pl.ds(c * NPAD + s * ROWS_T, ROWS_T)])


_sc_agg = functools.partial(
    pl.kernel,
    out_type=jax.ShapeDtypeStruct((NC * NPAD, D_OUT), jnp.float32),
    mesh=plsc.VectorSubcoreMesh(core_axis_name="c", subcore_axis_name="s",
                                num_cores=NC, num_subcores=NS),
    scratch_types=[
        pltpu.VMEM_SHARED((NPAD, D_OUT), jnp.float32),
        pltpu.VMEM((META,), jnp.int32),
        pltpu.VMEM((META,), jnp.int32),
        pltpu.VMEM((META,), jnp.int32),
        pltpu.VMEM((META,), jnp.float32),
        pltpu.VMEM((CHUNK,), jnp.int32),
        pltpu.VMEM((CHUNK,), jnp.int32),
        pltpu.VMEM((CHUNK,), jnp.int32),
        pltpu.VMEM((CHUNK,), jnp.int32),
        pltpu.VMEM((CHUNK, D_OUT), jnp.float32),
        pltpu.VMEM((CHUNK, D_OUT), jnp.float32),
        pltpu.VMEM((CHUNK, D_OUT), jnp.float32),
        pltpu.SemaphoreType.DMA,
        pltpu.SemaphoreType.DMA,
        pltpu.SemaphoreType.DMA,
        pltpu.SemaphoreType.DMA,
    ],
)(_sc_body)


def _combine_body(p_ref, b_ref, o_ref):
    o_ref[...] = p_ref[0] + p_ref[1] + b_ref[...]


def _combine(parts, b):
    BN = 2048
    NB = NPAD // BN
    return pl.pallas_call(
        _combine_body,
        grid=(NB,),
        in_specs=[
            pl.BlockSpec((NC, BN, D_OUT), lambda i: (0, i, 0)),
            pl.BlockSpec((1, D_OUT), lambda i: (0, 0)),
        ],
        out_specs=pl.BlockSpec((BN, D_OUT), lambda i: (i, 0)),
        out_shape=jax.ShapeDtypeStruct((NPAD, D_OUT), jnp.float32),
    )(parts, b)


def kernel(h, edge_index, edge_relation, edge_norm, W, b):
    hw = _hw_table(h, W).reshape(NUM_REL * N, D_OUT)
    row = edge_index[0]
    col = edge_index[1]
    parts = _sc_agg(hw, row, col, edge_relation, edge_norm)
    out = _combine(parts.reshape(NC, NPAD, D_OUT), b.reshape(1, D_OUT))
    return out[:N]
